# Initial kernel scaffold; baseline (speedup 1.0000x reference)
import jax
import jax.numpy as jnp
from jax.experimental import pallas as pl


def kernel(W_ctx, W_word, pos_context, pos_word, neg_word):
    raise NotImplementedError("write your pallas kernel here")


# speedup vs baseline: 4.9494x; 4.9494x over previous
import functools

import jax
import jax.numpy as jnp
from jax import lax
from jax.experimental import pallas as pl
from jax.experimental.pallas import tpu as pltpu
from jax.experimental.pallas import tpu_sc as plsc

_EMB_SIZE = 1000000
_EMB_DIM = 64
_B = 16384
_CTX = 20
_NC = 2
_NS = 16
_NW = _NC * _NS
_BPW = _B // _NW
_E = 16
_CHUNKS = _BPW // _E
_RPC = _E * _CTX
_IPW = _BPW * _CTX

_LN2 = 0.6931471805599453
_SQRT2 = 1.4142135623730951


def _log_pos(a):
    i = lax.bitcast_convert_type(a, jnp.int32)
    e = lax.shift_right_arithmetic(i, 23) - 127
    m = lax.bitcast_convert_type(
        jnp.bitwise_or(jnp.bitwise_and(i, 0x007FFFFF), 0x3F800000), jnp.float32)
    big = m > _SQRT2
    m = jnp.where(big, m * 0.5, m)
    e = jnp.where(big, e + 1, e)
    t = (m - 1.0) / (m + 1.0)
    t2 = t * t
    series = 1.0 + t2 * (1.0 / 3.0 + t2 * (1.0 / 5.0 + t2 * (
        1.0 / 7.0 + t2 * (1.0 / 9.0 + t2 * (1.0 / 11.0)))))
    return e.astype(jnp.float32) * _LN2 + 2.0 * t * series


def _lane_sum_splat(v):
    idx = jnp.arange(16, dtype=jnp.int32)
    dnums = lax.GatherDimensionNumbers(
        offset_dims=(), collapsed_slice_dims=(0,), start_index_map=(0,))
    for s in (1, 2, 4, 8):
        perm = jnp.bitwise_xor(idx, s)
        v = v + lax.gather(v, perm[:, None], dimension_numbers=dnums,
                           slice_sizes=(1,),
                           mode=lax.GatherScatterMode.PROMISE_IN_BOUNDS)
    return v


def _logsigmoid(x):
    return -_log_pos(1.0 + jnp.exp(-x))


def _sc_body(w_ctx, w_word, ctx_idx, pw_idx, neg_idx, out,
             idx_ctx_o, idx_neg_o, idx_pw_o,
             idx_ctx_p, idx_neg_p, idx_pw_p,
             rows_ctx_v, rows_neg_v, rows_pw_v, out_v, sem):
    wid = lax.axis_index("s") * _NC + lax.axis_index("c")

    pltpu.sync_copy(ctx_idx.at[pl.ds(wid * _IPW, _IPW)], idx_ctx_o)
    pltpu.sync_copy(neg_idx.at[pl.ds(wid * _IPW, _IPW)], idx_neg_o)
    pltpu.sync_copy(pw_idx.at[pl.ds(wid * _BPW, _BPW)],
                    idx_pw_o.at[pl.ds(0, _BPW)])

    def cv_body(i, carry):
        idx_ctx_p[pl.ds(i * 16, 16)] = lax.shift_right_logical(
            idx_ctx_o[pl.ds(i * 16, 16)], 1)
        idx_neg_p[pl.ds(i * 16, 16)] = lax.shift_right_logical(
            idx_neg_o[pl.ds(i * 16, 16)], 1)
        return carry

    lax.fori_loop(0, _IPW // 16, cv_body, 0, unroll=False)

    def cvp_body(i, carry):
        idx_pw_p[pl.ds(i * 16, 16)] = lax.shift_right_logical(
            idx_pw_o[pl.ds(i * 16, 16)], 1)
        return carry

    lax.fori_loop(0, _BPW // 16, cvp_body, 0, unroll=False)

    def chunk_body(c, acc):
        copies = []
        for start, size in ((0, 128), (128, 128), (256, 64)):
            copies.append(pltpu.async_copy(
                w_ctx.at[idx_ctx_p.at[pl.ds(c * _RPC + start, size)]],
                rows_ctx_v.at[pl.ds(start, size)], sem))
            copies.append(pltpu.async_copy(
                w_word.at[idx_neg_p.at[pl.ds(c * _RPC + start, size)]],
                rows_neg_v.at[pl.ds(start, size)], sem))
        copies.append(pltpu.async_copy(
            w_word.at[idx_pw_p.at[pl.ds(c * _E, _E)]], rows_pw_v, sem))
        for cp in copies:
            cp.wait()

        def ex_body(e, acc2):
            r0 = e * _CTX
            g0 = c * _RPC + r0
            oc1 = jnp.bitwise_and(idx_ctx_o[pl.ds(g0, 16)], 1) * 64
            oc2 = jnp.bitwise_and(idx_ctx_o[pl.ds(g0 + 4, 16)], 1) * 64
            on1 = jnp.bitwise_and(idx_neg_o[pl.ds(g0, 16)], 1) * 64
            on2 = jnp.bitwise_and(idx_neg_o[pl.ds(g0 + 4, 16)], 1) * 64

            def offc(j):
                return oc1[j] if j < 16 else oc2[j - 4]

            def offn(j):
                return on1[j] if j < 16 else on2[j - 4]

            a = [rows_ctx_v[r0, pl.ds(offc(0) + dc * 16, 16)] for dc in range(4)]
            nacc = [rows_neg_v[r0, pl.ds(offn(0) + dc * 16, 16)]
                    for dc in range(4)]
            for j in range(1, _CTX):
                oj, nj = offc(j), offn(j)
                for dc in range(4):
                    a[dc] = a[dc] + rows_ctx_v[r0 + j, pl.ds(oj + dc * 16, 16)]
                    nacc[dc] = nacc[dc] + rows_neg_v[r0 + j,
                                                     pl.ds(nj + dc * 16, 16)]
            opv = jnp.bitwise_and(idx_pw_o[pl.ds(c * _E + e, 16)], 1) * 64
            offp = opv[0]
            pvec = [rows_pw_v[e, pl.ds(offp + dc * 16, 16)] for dc in range(4)]
            sp = a[0] * pvec[0] + a[1] * pvec[1] + a[2] * pvec[2] + a[3] * pvec[3]
            sn = a[0] * nacc[0] + a[1] * nacc[1] + a[2] * nacc[2] + a[3] * nacc[3]
            spl_sp = _lane_sum_splat(sp)
            spl_sn = _lane_sum_splat(sn)
            x = jnp.where(jnp.arange(16, dtype=jnp.int32) < 8, spl_sp, -spl_sn)
            return acc2 + _logsigmoid(x)

        return lax.fori_loop(0, _E, ex_body, acc, unroll=False)

    accv = lax.fori_loop(0, _CHUNKS, chunk_body,
                         jnp.zeros((16,), jnp.float32), unroll=False)
    total = _lane_sum_splat(accv) * 0.125
    out_v[0, :] = jnp.where(jnp.arange(16, dtype=jnp.int32) == 0, total, 0.0)
    zeros = jnp.zeros((16,), jnp.float32)
    for r in range(1, 8):
        out_v[r, :] = zeros
    pltpu.sync_copy(out_v, out.at[wid])


@jax.jit
def _cbow_loss_sc(w_ctx, w_word, ctx_idx, pw_idx, neg_idx):
    mesh = plsc.VectorSubcoreMesh(core_axis_name="c", subcore_axis_name="s")
    kfn = functools.partial(
        pl.kernel, mesh=mesh,
        out_type=jax.ShapeDtypeStruct((_NW, 8, 16), jnp.float32),
        scratch_types=[
            pltpu.VMEM((_IPW,), jnp.int32),
            pltpu.VMEM((_IPW,), jnp.int32),
            pltpu.VMEM((_BPW + 16,), jnp.int32),
            pltpu.VMEM((_IPW,), jnp.int32),
            pltpu.VMEM((_IPW,), jnp.int32),
            pltpu.VMEM((_BPW,), jnp.int32),
            pltpu.VMEM((_RPC, 2 * _EMB_DIM), jnp.float32),
            pltpu.VMEM((_RPC, 2 * _EMB_DIM), jnp.float32),
            pltpu.VMEM((_E, 2 * _EMB_DIM), jnp.float32),
            pltpu.VMEM((8, 16), jnp.float32),
            pltpu.SemaphoreType.DMA,
        ],
    )(_sc_body)
    return kfn(w_ctx, w_word, ctx_idx, pw_idx, neg_idx)


def kernel(W_ctx, W_word, pos_context, pos_word, neg_word):
    w_ctx2 = W_ctx.reshape(_EMB_SIZE // 2, 2 * _EMB_DIM)
    w_word2 = W_word.reshape(_EMB_SIZE // 2, 2 * _EMB_DIM)
    ctx_idx = pos_context.reshape(-1)
    neg_idx = neg_word.reshape(-1)
    partials = _cbow_loss_sc(w_ctx2, w_word2, ctx_idx, pos_word, neg_idx)
    return -jnp.sum(partials)
